# baseline (speedup 1.0000x reference)
import functools

import jax
import jax.numpy as jnp
from jax import lax
from jax.experimental import pallas as pl
from jax.experimental.pallas import tpu as pltpu
from jax.experimental.pallas import tpu_sc as plsc

N = 10000
F = 64
K = 120
KP = 128
NW = 32
VPW = 320
NP = NW * VPW
NF16 = F // 16
CV = 80

_BCAST_DNUMS = lax.GatherDimensionNumbers(
    offset_dims=(), collapsed_slice_dims=(0,), start_index_map=(0,))


def _bcast_lane(vec, l):
    return lax.gather(vec, jnp.full((16, 1), l, jnp.int32), _BCAST_DNUMS,
                      slice_sizes=(1,),
                      mode=lax.GatherScatterMode.PROMISE_IN_BOUNDS)


def _sc_interp_body(signal_hbm, idx_hbm, w_hbm, out_hbm,
                    idx_c, w_c, rows_v, s_v, table_sh, sem0, sem1):
    cid = lax.axis_index("c")
    sid = lax.axis_index("s")
    wid = sid * 2 + cid
    base = wid * VPW

    tps = NP // 16
    pltpu.sync_copy(signal_hbm.at[pl.ds(sid * tps, tps)],
                    table_sh.at[pl.ds(sid * tps, tps)])
    plsc.subcore_barrier()

    sems = (sem0, sem1)

    def fire(i, b):
        pltpu.async_copy(table_sh.at[idx_c.at[i]], rows_v.at[b], sems[b])

    def wait(b):
        pltpu.make_async_copy(signal_hbm.at[pl.ds(0, KP)],
                              rows_v.at[b], sems[b]).wait()

    def compute(gi, i, b):
        rows = rows_v.at[b]
        accs = [jnp.zeros((16,), jnp.float32) for _ in range(NF16)]
        for j0 in range(0, KP, 16):
            wv = w_c[i, pl.ds(j0, 16)]
            for l in range(16):
                j = j0 + l
                wb = _bcast_lane(wv, l)
                for f in range(NF16):
                    accs[f] = accs[f] + wb * rows[j, pl.ds(16 * f, 16)]
        for f in range(NF16):
            s_v[gi, pl.ds(16 * f, 16)] = accs[f]

    def chunk_body(c, carry):
        c0 = c * CV
        pltpu.sync_copy(idx_hbm.at[pl.ds(base + c0, CV)], idx_c)
        pltpu.sync_copy(w_hbm.at[pl.ds(base + c0, CV)], w_c)
        fire(0, 0)
        fire(1, 1)

        def body(it, carry2):
            i0 = 2 * it
            wait(0)
            compute(c0 + i0, i0, 0)

            @pl.when(i0 + 2 < CV)
            def _():
                fire(i0 + 2, 0)

            wait(1)
            compute(c0 + i0 + 1, i0 + 1, 1)

            @pl.when(i0 + 3 < CV)
            def _():
                fire(i0 + 3, 1)

            return carry2

        lax.fori_loop(0, CV // 2, body, 0)
        return carry

    lax.fori_loop(0, VPW // CV, chunk_body, 0)
    pltpu.sync_copy(s_v, out_hbm.at[pl.ds(base, VPW)])


_sc_interp = functools.partial(
    pl.kernel,
    out_type=jax.ShapeDtypeStruct((NP, F), jnp.float32),
    mesh=plsc.VectorSubcoreMesh(core_axis_name="c", subcore_axis_name="s"),
    compiler_params=pltpu.CompilerParams(use_tc_tiling_on_sc=False),
    scratch_types=[
        pltpu.VMEM((CV, KP), jnp.int32),
        pltpu.VMEM((CV, KP), jnp.float32),
        pltpu.VMEM((2, KP, F), jnp.float32),
        pltpu.VMEM((VPW, F), jnp.float32),
        pltpu.VMEM_SHARED((NP, F), jnp.float32),
        pltpu.SemaphoreType.DMA,
        pltpu.SemaphoreType.DMA,
    ],
)(_sc_interp_body)


def _fold_body(s_ref, wt_ref, b_ref, o_ref):
    s = s_ref[...]
    y0 = jnp.dot(s, wt_ref[0], preferred_element_type=jnp.float32) + b_ref[0]
    y1 = jnp.dot(s, wt_ref[1], preferred_element_type=jnp.float32) + b_ref[1]
    o_ref[...] = jnp.maximum(y0, 0.0) + jnp.maximum(y1, 0.0)


def kernel(signal, bary_coordinates, template_weights, bias):
    idx = bary_coordinates[..., 0].astype(jnp.int32).reshape(N, K)
    w = bary_coordinates[..., 1].reshape(N, K)
    idx_p = jnp.pad(idx, ((0, NP - N), (0, KP - K)))
    w_p = jnp.pad(w, ((0, NP - N), (0, KP - K)))
    signal_p = jnp.pad(signal, ((0, NP - N), (0, 0)))

    s = _sc_interp(signal_p, idx_p, w_p)

    wt = jnp.transpose(template_weights, (0, 2, 1))
    bias_p = jnp.pad(bias, ((0, 6), (0, 0)))

    bn = NP // 8
    out = pl.pallas_call(
        _fold_body,
        grid=(8,),
        in_specs=[
            pl.BlockSpec((bn, F), lambda i: (i, 0)),
            pl.BlockSpec((2, F, F), lambda i: (0, 0, 0)),
            pl.BlockSpec((8, F), lambda i: (0, 0)),
        ],
        out_specs=pl.BlockSpec((bn, F), lambda i: (i, 0)),
        out_shape=jax.ShapeDtypeStruct((NP, F), jnp.float32),
    )(s, wt, bias_p)
    return out[:N]


# speedup vs baseline: 1.3561x; 1.3561x over previous
import functools

import jax
import jax.numpy as jnp
from jax import lax
from jax.experimental import pallas as pl
from jax.experimental.pallas import tpu as pltpu
from jax.experimental.pallas import tpu_sc as plsc

N = 10000
F = 64
K = 120
KP = 128
NW = 32
VPW = 320
NP = NW * VPW
NF16 = F // 16
CV = 80

_BCAST_DNUMS = lax.GatherDimensionNumbers(
    offset_dims=(), collapsed_slice_dims=(0,), start_index_map=(0,))


def _bcast_lane(vec, l):
    return lax.gather(vec, jnp.full((16, 1), l, jnp.int32), _BCAST_DNUMS,
                      slice_sizes=(1,),
                      mode=lax.GatherScatterMode.PROMISE_IN_BOUNDS)


def _sc_interp_body(signal_hbm, idx_hbm, w_hbm, out_hbm,
                    idx_c, w_c, rows_v, s_v, table_sh, sem0, sem1):
    cid = lax.axis_index("c")
    sid = lax.axis_index("s")
    wid = sid * 2 + cid
    base = wid * VPW

    tps = NP // 16
    pltpu.sync_copy(signal_hbm.at[pl.ds(sid * tps, tps)],
                    table_sh.at[pl.ds(sid * tps, tps)])
    plsc.subcore_barrier()

    sems = (sem0, sem1)

    def fire(i, b):
        pltpu.async_copy(table_sh.at[idx_c.at[i]], rows_v.at[b], sems[b])

    def wait(b):
        pltpu.make_async_copy(signal_hbm.at[pl.ds(0, KP)],
                              rows_v.at[b], sems[b]).wait()

    def compute(gi, i, b):
        rows = rows_v.at[b]
        acc_a = [jnp.zeros((16,), jnp.float32) for _ in range(NF16)]
        acc_b = [jnp.zeros((16,), jnp.float32) for _ in range(NF16)]
        for j0 in range(0, KP, 16):
            wv = w_c[i, pl.ds(j0, 16)]
            for l in range(0, 16, 2):
                wa = wv[l]
                wb = wv[l + 1]
                j = j0 + l
                for f in range(NF16):
                    acc_a[f] = acc_a[f] + wa * rows[j, pl.ds(16 * f, 16)]
                    acc_b[f] = acc_b[f] + wb * rows[j + 1, pl.ds(16 * f, 16)]
        for f in range(NF16):
            s_v[gi, pl.ds(16 * f, 16)] = acc_a[f] + acc_b[f]

    def chunk_body(c, carry):
        c0 = c * CV
        pltpu.sync_copy(idx_hbm.at[pl.ds(base + c0, CV)], idx_c)
        pltpu.sync_copy(w_hbm.at[pl.ds(base + c0, CV)], w_c)
        fire(0, 0)
        fire(1, 1)

        def body(it, carry2):
            i0 = 2 * it
            wait(0)
            compute(c0 + i0, i0, 0)

            @pl.when(i0 + 2 < CV)
            def _():
                fire(i0 + 2, 0)

            wait(1)
            compute(c0 + i0 + 1, i0 + 1, 1)

            @pl.when(i0 + 3 < CV)
            def _():
                fire(i0 + 3, 1)

            return carry2

        lax.fori_loop(0, CV // 2, body, 0)
        return carry

    lax.fori_loop(0, VPW // CV, chunk_body, 0)
    pltpu.sync_copy(s_v, out_hbm.at[pl.ds(base, VPW)])


_sc_interp = functools.partial(
    pl.kernel,
    out_type=jax.ShapeDtypeStruct((NP, F), jnp.float32),
    mesh=plsc.VectorSubcoreMesh(core_axis_name="c", subcore_axis_name="s"),
    compiler_params=pltpu.CompilerParams(use_tc_tiling_on_sc=False),
    scratch_types=[
        pltpu.VMEM((CV, KP), jnp.int32),
        pltpu.VMEM((CV, KP), jnp.float32),
        pltpu.VMEM((2, KP, F), jnp.float32),
        pltpu.VMEM((VPW, F), jnp.float32),
        pltpu.VMEM_SHARED((NP, F), jnp.float32),
        pltpu.SemaphoreType.DMA,
        pltpu.SemaphoreType.DMA,
    ],
)(_sc_interp_body)


def _fold_body(s_ref, wt_ref, b_ref, o_ref):
    s = s_ref[...]
    y0 = jnp.dot(s, wt_ref[0], preferred_element_type=jnp.float32) + b_ref[0]
    y1 = jnp.dot(s, wt_ref[1], preferred_element_type=jnp.float32) + b_ref[1]
    o_ref[...] = jnp.maximum(y0, 0.0) + jnp.maximum(y1, 0.0)


def kernel(signal, bary_coordinates, template_weights, bias):
    idx = bary_coordinates[..., 0].astype(jnp.int32).reshape(N, K)
    w = bary_coordinates[..., 1].reshape(N, K)
    idx_p = jnp.pad(idx, ((0, NP - N), (0, KP - K)))
    w_p = jnp.pad(w, ((0, NP - N), (0, KP - K)))
    signal_p = jnp.pad(signal, ((0, NP - N), (0, 0)))

    s = _sc_interp(signal_p, idx_p, w_p)

    wt = jnp.transpose(template_weights, (0, 2, 1))
    bias_p = jnp.pad(bias, ((0, 6), (0, 0)))

    bn = NP // 8
    out = pl.pallas_call(
        _fold_body,
        grid=(8,),
        in_specs=[
            pl.BlockSpec((bn, F), lambda i: (i, 0)),
            pl.BlockSpec((2, F, F), lambda i: (0, 0, 0)),
            pl.BlockSpec((8, F), lambda i: (0, 0)),
        ],
        out_specs=pl.BlockSpec((bn, F), lambda i: (i, 0)),
        out_shape=jax.ShapeDtypeStruct((NP, F), jnp.float32),
    )(s, wt, bias_p)
    return out[:N]


# speedup vs baseline: 1.4762x; 1.0886x over previous
import functools

import jax
import jax.numpy as jnp
from jax import lax
from jax.experimental import pallas as pl
from jax.experimental.pallas import tpu as pltpu
from jax.experimental.pallas import tpu_sc as plsc

N = 10000
F = 64
K = 120
KP = 128
NW = 32
VPW = 320
NP = NW * VPW
NF16 = F // 16
CV = 80

_BCAST_DNUMS = lax.GatherDimensionNumbers(
    offset_dims=(), collapsed_slice_dims=(0,), start_index_map=(0,))


def _bcast_lane(vec, l):
    return lax.gather(vec, jnp.full((16, 1), l, jnp.int32), _BCAST_DNUMS,
                      slice_sizes=(1,),
                      mode=lax.GatherScatterMode.PROMISE_IN_BOUNDS)


def _sc_interp_body(signal_hbm, idx_hbm, w_hbm, out_hbm,
                    idx_c, w_c, rows_v, s_v, table_sh, sem0, sem1):
    cid = lax.axis_index("c")
    sid = lax.axis_index("s")
    wid = sid * 2 + cid
    base = wid * VPW

    tps = NP // 16
    pltpu.sync_copy(signal_hbm.at[pl.ds(sid * tps, tps)],
                    table_sh.at[pl.ds(sid * tps, tps)])
    plsc.subcore_barrier()

    sems = (sem0, sem1)

    def fire(i, b):
        pltpu.async_copy(table_sh.at[idx_c.at[i]], rows_v.at[b], sems[b])

    def wait(b):
        pltpu.make_async_copy(signal_hbm.at[pl.ds(0, KP)],
                              rows_v.at[b], sems[b]).wait()

    def compute(gi, i, b):
        rows = rows_v.at[b]

        def jchunk(jc, accs):
            j0 = jc * 16
            wv = w_c[i, pl.ds(j0, 16)]
            acc_a, acc_b = list(accs[:NF16]), list(accs[NF16:])
            for l in range(0, 16, 2):
                wa = wv[l]
                wb = wv[l + 1]
                for f in range(NF16):
                    acc_a[f] = acc_a[f] + wa * rows[j0 + l, pl.ds(16 * f, 16)]
                    acc_b[f] = (acc_b[f]
                                + wb * rows[j0 + l + 1, pl.ds(16 * f, 16)])
            return tuple(acc_a) + tuple(acc_b)

        zeros = tuple(jnp.zeros((16,), jnp.float32) for _ in range(2 * NF16))
        accs = lax.fori_loop(0, KP // 16, jchunk, zeros)
        for f in range(NF16):
            s_v[gi, pl.ds(16 * f, 16)] = accs[f] + accs[NF16 + f]

    def chunk_body(c, carry):
        c0 = c * CV
        pltpu.sync_copy(idx_hbm.at[pl.ds(base + c0, CV)], idx_c)
        pltpu.sync_copy(w_hbm.at[pl.ds(base + c0, CV)], w_c)
        fire(0, 0)
        fire(1, 1)

        def body(it, carry2):
            buf = it & 1

            @pl.when(buf == 0)
            def _():
                wait(0)

            @pl.when(buf == 1)
            def _():
                wait(1)

            compute(c0 + it, it, buf)

            @pl.when(jnp.logical_and(it + 2 < CV, buf == 0))
            def _():
                fire(it + 2, 0)

            @pl.when(jnp.logical_and(it + 2 < CV, buf == 1))
            def _():
                fire(it + 2, 1)

            return carry2

        lax.fori_loop(0, CV, body, 0)
        return carry

    lax.fori_loop(0, VPW // CV, chunk_body, 0)
    pltpu.sync_copy(s_v, out_hbm.at[pl.ds(base, VPW)])


_sc_interp = functools.partial(
    pl.kernel,
    out_type=jax.ShapeDtypeStruct((NP, F), jnp.float32),
    mesh=plsc.VectorSubcoreMesh(core_axis_name="c", subcore_axis_name="s"),
    compiler_params=pltpu.CompilerParams(use_tc_tiling_on_sc=False),
    scratch_types=[
        pltpu.VMEM((CV, KP), jnp.int32),
        pltpu.VMEM((CV, KP), jnp.float32),
        pltpu.VMEM((2, KP, F), jnp.float32),
        pltpu.VMEM((VPW, F), jnp.float32),
        pltpu.VMEM_SHARED((NP, F), jnp.float32),
        pltpu.SemaphoreType.DMA,
        pltpu.SemaphoreType.DMA,
    ],
)(_sc_interp_body)


def _fold_body(s_ref, wt_ref, b_ref, o_ref):
    s = s_ref[...]
    y0 = jnp.dot(s, wt_ref[0], preferred_element_type=jnp.float32) + b_ref[0]
    y1 = jnp.dot(s, wt_ref[1], preferred_element_type=jnp.float32) + b_ref[1]
    o_ref[...] = jnp.maximum(y0, 0.0) + jnp.maximum(y1, 0.0)


def kernel(signal, bary_coordinates, template_weights, bias):
    idx = bary_coordinates[..., 0].astype(jnp.int32).reshape(N, K)
    w = bary_coordinates[..., 1].reshape(N, K)
    idx_p = jnp.pad(idx, ((0, NP - N), (0, KP - K)))
    w_p = jnp.pad(w, ((0, NP - N), (0, KP - K)))
    signal_p = jnp.pad(signal, ((0, NP - N), (0, 0)))

    s = _sc_interp(signal_p, idx_p, w_p)

    wt = jnp.transpose(template_weights, (0, 2, 1))
    bias_p = jnp.pad(bias, ((0, 6), (0, 0)))

    bn = NP // 8
    out = pl.pallas_call(
        _fold_body,
        grid=(8,),
        in_specs=[
            pl.BlockSpec((bn, F), lambda i: (i, 0)),
            pl.BlockSpec((2, F, F), lambda i: (0, 0, 0)),
            pl.BlockSpec((8, F), lambda i: (0, 0)),
        ],
        out_specs=pl.BlockSpec((bn, F), lambda i: (i, 0)),
        out_shape=jax.ShapeDtypeStruct((NP, F), jnp.float32),
    )(s, wt, bias_p)
    return out[:N]
